# baseline (speedup 1.0000x reference)
import functools

import jax
import jax.numpy as jnp
from jax import lax
from jax.experimental import pallas as pl
from jax.experimental.pallas import tpu as pltpu
from jax.experimental.pallas import tpu_sc as plsc

NC = 2
NS = 16
L = 16
NW = NC * NS
C = 128


def _sqrt16(x):
    i = lax.bitcast_convert_type(x, jnp.int32)
    i = jnp.int32(0x5F3759DF) - lax.shift_right_logical(i, 1)
    y = lax.bitcast_convert_type(i, jnp.float32)
    half = x * 0.5
    for _ in range(3):
        y = y * (1.5 - half * y * y)
    return x * y


def _lo(word):
    return lax.bitcast_convert_type(lax.shift_left(word, 16), jnp.float32)


def _hi(word):
    return lax.bitcast_convert_type(
        lax.bitwise_and(word, jnp.int32(-65536)), jnp.float32)


def _pack(x):
    u = lax.bitcast_convert_type(x.astype(jnp.bfloat16), jnp.uint16)
    u = u.astype(jnp.uint32)
    word = u[:, ::2] | (u[:, 1::2] << 16)
    return lax.bitcast_convert_type(word, jnp.int32)


def kernel(h_batch, t_batch, l_batch, h_apos_batch, t_apos_batch,
           l_apos_batch, E, R, W):
    B = h_batch.shape[0]
    D = E.shape[1]
    NR = R.shape[0]
    DP = D // 2
    f32 = jnp.float32
    b_per_w = B // NW
    nchunk = b_per_w // C
    assert b_per_w * NW == B and nchunk * C == b_per_w and D % (2 * L) == 0

    mesh = plsc.VectorSubcoreMesh(core_axis_name="c", subcore_axis_name="s")
    vec = jax.ShapeDtypeStruct((B,), f32)
    matT = jax.ShapeDtypeStruct((D, B), f32)

    idx_t = pltpu.VMEM((C,), jnp.int32)
    row_t = pltpu.VMEM((C, DP), jnp.int32)
    out_t = pltpu.VMEM((D, C), f32)

    @functools.partial(
        pl.kernel,
        out_type=(vec, vec, matT, matT, matT, matT),
        mesh=mesh,
        compiler_params=pltpu.CompilerParams(
            needs_layout_passes=False, use_tc_tiling_on_sc=False),
        scratch_types=[
            [idx_t] * 2, [idx_t] * 2, [idx_t] * 2,
            [row_t] * 2, [row_t] * 2,
            pltpu.VMEM((NR, DP), jnp.int32),
            pltpu.VMEM((NR, DP), jnp.int32),
            [out_t] * 2, [out_t] * 2,
            [pltpu.VMEM((C,), f32)] * 2,
            pltpu.SemaphoreType.DMA,
            pltpu.SemaphoreType.DMA,
        ],
    )
    def run(h_i, t_i, l_i, ha_i, ta_i, la_i, E_h, R_h, W_h,
            dist_o, dista_o, hp_o, tp_o, hpa_o, tpa_o,
            hi_v, ti_v, li_v, hr, tr, R_t, W_t, hp_v, tp_v, dist_v,
            gsem, ssem):
        cid = lax.axis_index("c")
        sid = lax.axis_index("s")
        wid = sid * NC + cid
        zero = jnp.zeros((L,), f32)

        pltpu.sync_copy(R_h, R_t)
        pltpu.sync_copy(W_h, W_t)

        sides = (
            (h_i, t_i, l_i, dist_o, hp_o, tp_o),
            (ha_i, ta_i, la_i, dista_o, hpa_o, tpa_o),
        )
        tasks = [(s, c) for s in range(2) for c in range(nchunk)]

        def start_gathers(task, slot):
            s, c = task
            hb, tb, lb, _, _, _ = sides[s]
            base = wid * b_per_w + c * C
            pltpu.sync_copy(hb.at[pl.ds(base, C)], hi_v[slot])
            pltpu.sync_copy(tb.at[pl.ds(base, C)], ti_v[slot])
            pltpu.sync_copy(lb.at[pl.ds(base, C)], li_v[slot])
            return [
                pltpu.async_copy(E_h.at[hi_v[slot]], hr[slot], gsem),
                pltpu.async_copy(E_h.at[ti_v[slot]], tr[slot], gsem),
            ]

        def compute(slot):
            hrs, trs, R32, W32 = hr[slot], tr[slot], R_t, W_t
            lis = li_v[slot]
            hps, tps, dv = hp_v[slot], tp_v[slot], dist_v[slot]

            @plsc.parallel_loop(0, C // L)
            def _group(g):
                rows = g * L + lax.iota(jnp.int32, L)
                lv = lis[pl.ds(g * L, L)]

                @plsc.parallel_loop(0, DP, unroll=8, carry=(zero, zero, zero))
                def dots(d, dcarry):
                    n2, sh, st = dcarry
                    col = jnp.full((L,), d, jnp.int32)
                    ww = plsc.load_gather(W32, [lv, col])
                    hw = plsc.load_gather(hrs, [rows, col])
                    tw = plsc.load_gather(trs, [rows, col])
                    w0, w1 = _lo(ww), _hi(ww)
                    h0, h1 = _lo(hw), _hi(hw)
                    t0, t1 = _lo(tw), _hi(tw)
                    return (n2 + w0 * w0 + w1 * w1,
                            sh + h0 * w0 + h1 * w1,
                            st + t0 * w0 + t1 * w1)

                n2, sh, st = dots
                ah = sh / n2
                atc = st / n2

                @plsc.parallel_loop(0, DP, unroll=8, carry=zero)
                def accd(d, acc):
                    col = jnp.full((L,), d, jnp.int32)
                    ww = plsc.load_gather(W32, [lv, col])
                    hw = plsc.load_gather(hrs, [rows, col])
                    tw = plsc.load_gather(trs, [rows, col])
                    lw = plsc.load_gather(R32, [lv, col])
                    w0, w1 = _lo(ww), _hi(ww)
                    h0, h1 = _lo(hw), _hi(hw)
                    t0, t1 = _lo(tw), _hi(tw)
                    l0, l1 = _lo(lw), _hi(lw)
                    hp0 = h0 - ah * w0
                    hp1 = h1 - ah * w1
                    tp0 = t0 - atc * w0
                    tp1 = t1 - atc * w1
                    hps[2 * d, pl.ds(g * L, L)] = hp0
                    hps[2 * d + 1, pl.ds(g * L, L)] = hp1
                    tps[2 * d, pl.ds(g * L, L)] = tp0
                    tps[2 * d + 1, pl.ds(g * L, L)] = tp1
                    q0 = hp0 + l0 - tp0
                    q1 = hp1 + l1 - tp1
                    return acc + q0 * q0 + q1 * q1

                dv[pl.ds(g * L, L)] = _sqrt16(accd)

        def start_stores(task, slot):
            s, c = task
            _, _, _, d_o, hpo, tpo = sides[s]
            base = wid * b_per_w + c * C
            return [
                pltpu.async_copy(hp_v[slot], hpo.at[:, pl.ds(base, C)], ssem),
                pltpu.async_copy(tp_v[slot], tpo.at[:, pl.ds(base, C)], ssem),
                pltpu.async_copy(dist_v[slot], d_o.at[pl.ds(base, C)], ssem),
            ]

        pending_g = start_gathers(tasks[0], 0)
        pending_s = []
        for i, task in enumerate(tasks):
            slot = i % 2
            for cp in pending_g:
                cp.wait()
            if i + 1 < len(tasks):
                pending_g = start_gathers(tasks[i + 1], (i + 1) % 2)
            for cp in pending_s:
                cp.wait()
            compute(slot)
            pending_s = start_stores(task, slot)
        for cp in pending_s:
            cp.wait()

    dist, dist_a, hpT, tpT, hpaT, tpaT = run(
        h_batch.astype(jnp.int32), t_batch.astype(jnp.int32),
        l_batch.astype(jnp.int32), h_apos_batch.astype(jnp.int32),
        t_apos_batch.astype(jnp.int32), l_apos_batch.astype(jnp.int32),
        _pack(E), _pack(R), _pack(W))
    return (dist, dist_a, jnp.transpose(hpT), jnp.transpose(tpT),
            jnp.transpose(hpaT), jnp.transpose(tpaT))


# speedup vs baseline: 12.6739x; 12.6739x over previous
import functools

import jax
import jax.numpy as jnp
from jax import lax
from jax.experimental import pallas as pl
from jax.experimental.pallas import tpu as pltpu
from jax.experimental.pallas import tpu_sc as plsc

NC = 2
NS = 16
L = 16
NW = NC * NS
C = 128


def _sqrt16(x):
    i = lax.bitcast_convert_type(x, jnp.int32)
    i = jnp.int32(0x5F3759DF) - lax.shift_right_logical(i, 1)
    y = lax.bitcast_convert_type(i, jnp.float32)
    half = x * 0.5
    for _ in range(3):
        y = y * (1.5 - half * y * y)
    return x * y


def _lo(word):
    return lax.bitcast_convert_type(lax.shift_left(word, 16), jnp.float32)


def _hi(word):
    return lax.bitcast_convert_type(
        lax.bitwise_and(word, jnp.int32(-65536)), jnp.float32)


def _pack(x):
    u = lax.bitcast_convert_type(x.astype(jnp.bfloat16), jnp.uint16)
    u = u.astype(jnp.uint32)
    word = u[:, ::2] | (u[:, 1::2] << 16)
    return lax.bitcast_convert_type(word, jnp.int32)


def kernel(h_batch, t_batch, l_batch, h_apos_batch, t_apos_batch,
           l_apos_batch, E, R, W):
    B = h_batch.shape[0]
    D = E.shape[1]
    NR = R.shape[0]
    DP = D // 2
    f32 = jnp.float32
    b_per_w = B // NW
    nchunk = b_per_w // C
    assert b_per_w * NW == B and nchunk * C == b_per_w and D % (2 * L) == 0

    mesh = plsc.VectorSubcoreMesh(core_axis_name="c", subcore_axis_name="s")
    vec = jax.ShapeDtypeStruct((B,), f32)
    matT = jax.ShapeDtypeStruct((D, B), f32)

    idx_t = pltpu.VMEM((C,), jnp.int32)
    row_t = pltpu.VMEM((C, D), f32)
    out_t = pltpu.VMEM((D, C), f32)

    @functools.partial(
        pl.kernel,
        out_type=(vec, vec, matT, matT, matT, matT),
        mesh=mesh,
        compiler_params=pltpu.CompilerParams(
            needs_layout_passes=False, use_tc_tiling_on_sc=False),
        scratch_types=[
            [idx_t] * 2, [idx_t] * 2, [idx_t] * 2,
            [row_t] * 2, [row_t] * 2,
            pltpu.VMEM((NR, DP), jnp.int32),
            pltpu.VMEM((NR, DP), jnp.int32),
            [out_t], [out_t],
            [pltpu.VMEM((C,), f32)],
            pltpu.SemaphoreType.DMA,
            pltpu.SemaphoreType.DMA,
        ],
    )
    def run(h_i, t_i, l_i, ha_i, ta_i, la_i, E_h, R_h, W_h,
            dist_o, dista_o, hp_o, tp_o, hpa_o, tpa_o,
            hi_v, ti_v, li_v, hr, tr, R_t, W_t, hp_v, tp_v, dist_v,
            gsem, ssem):
        cid = lax.axis_index("c")
        sid = lax.axis_index("s")
        wid = sid * NC + cid
        zero = jnp.zeros((L,), f32)

        pltpu.sync_copy(R_h, R_t)
        pltpu.sync_copy(W_h, W_t)

        sides = (
            (h_i, t_i, l_i, dist_o, hp_o, tp_o),
            (ha_i, ta_i, la_i, dista_o, hpa_o, tpa_o),
        )
        tasks = [(s, c) for s in range(2) for c in range(nchunk)]

        def start_gathers(task, slot):
            s, c = task
            hb, tb, lb, _, _, _ = sides[s]
            base = wid * b_per_w + c * C
            pltpu.sync_copy(hb.at[pl.ds(base, C)], hi_v[slot])
            pltpu.sync_copy(tb.at[pl.ds(base, C)], ti_v[slot])
            pltpu.sync_copy(lb.at[pl.ds(base, C)], li_v[slot])
            return [
                pltpu.async_copy(E_h.at[hi_v[slot]], hr[slot], gsem),
                pltpu.async_copy(E_h.at[ti_v[slot]], tr[slot], gsem),
            ]

        def compute(slot):
            hrs, trs, R32, W32 = hr[slot], tr[slot], R_t, W_t
            lis = li_v[slot]
            hps, tps, dv = hp_v[0], tp_v[0], dist_v[0]

            @plsc.parallel_loop(0, C // L)
            def _group(g):
                rows = g * L + lax.iota(jnp.int32, L)
                lv = lis[pl.ds(g * L, L)]

                @plsc.parallel_loop(0, DP, unroll=8, carry=(zero, zero, zero))
                def dots(d, dcarry):
                    n2, sh, st = dcarry
                    col = jnp.full((L,), d, jnp.int32)
                    d2 = 2 * d
                    c0 = jnp.full((L,), 0, jnp.int32) + d2
                    c1 = c0 + 1
                    ww = plsc.load_gather(W32, [lv, col])
                    h0 = plsc.load_gather(hrs, [rows, c0])
                    h1 = plsc.load_gather(hrs, [rows, c1])
                    t0 = plsc.load_gather(trs, [rows, c0])
                    t1 = plsc.load_gather(trs, [rows, c1])
                    w0, w1 = _lo(ww), _hi(ww)
                    return (n2 + w0 * w0 + w1 * w1,
                            sh + h0 * w0 + h1 * w1,
                            st + t0 * w0 + t1 * w1)

                n2, sh, st = dots
                ah = sh / n2
                atc = st / n2

                @plsc.parallel_loop(0, DP, unroll=8, carry=zero)
                def accd(d, acc):
                    col = jnp.full((L,), d, jnp.int32)
                    d2 = 2 * d
                    c0 = jnp.full((L,), 0, jnp.int32) + d2
                    c1 = c0 + 1
                    ww = plsc.load_gather(W32, [lv, col])
                    lw = plsc.load_gather(R32, [lv, col])
                    h0 = plsc.load_gather(hrs, [rows, c0])
                    h1 = plsc.load_gather(hrs, [rows, c1])
                    t0 = plsc.load_gather(trs, [rows, c0])
                    t1 = plsc.load_gather(trs, [rows, c1])
                    w0, w1 = _lo(ww), _hi(ww)
                    l0, l1 = _lo(lw), _hi(lw)
                    hp0 = h0 - ah * w0
                    hp1 = h1 - ah * w1
                    tp0 = t0 - atc * w0
                    tp1 = t1 - atc * w1
                    hps[d2, pl.ds(g * L, L)] = hp0
                    hps[d2 + 1, pl.ds(g * L, L)] = hp1
                    tps[d2, pl.ds(g * L, L)] = tp0
                    tps[d2 + 1, pl.ds(g * L, L)] = tp1
                    q0 = hp0 + l0 - tp0
                    q1 = hp1 + l1 - tp1
                    return acc + q0 * q0 + q1 * q1

                dv[pl.ds(g * L, L)] = _sqrt16(accd)

        def start_stores(task, slot):
            s, c = task
            _, _, _, d_o, hpo, tpo = sides[s]
            base = wid * b_per_w + c * C
            return [
                pltpu.async_copy(hp_v[0], hpo.at[:, pl.ds(base, C)], ssem),
                pltpu.async_copy(tp_v[0], tpo.at[:, pl.ds(base, C)], ssem),
                pltpu.async_copy(dist_v[0], d_o.at[pl.ds(base, C)], ssem),
            ]

        pending_g = start_gathers(tasks[0], 0)
        pending_s = []
        for i, task in enumerate(tasks):
            slot = i % 2
            for cp in pending_g:
                cp.wait()
            if i + 1 < len(tasks):
                pending_g = start_gathers(tasks[i + 1], (i + 1) % 2)
            for cp in pending_s:
                cp.wait()
            compute(slot)
            pending_s = start_stores(task, slot)
        for cp in pending_s:
            cp.wait()

    dist, dist_a, hpT, tpT, hpaT, tpaT = run(
        h_batch.astype(jnp.int32), t_batch.astype(jnp.int32),
        l_batch.astype(jnp.int32), h_apos_batch.astype(jnp.int32),
        t_apos_batch.astype(jnp.int32), l_apos_batch.astype(jnp.int32),
        E, _pack(R), _pack(W))
    return (dist, dist_a, jnp.transpose(hpT), jnp.transpose(tpT),
            jnp.transpose(hpaT), jnp.transpose(tpaT))


# speedup vs baseline: 12.7027x; 1.0023x over previous
import functools

import jax
import jax.numpy as jnp
from jax import lax
from jax.experimental import pallas as pl
from jax.experimental.pallas import tpu as pltpu
from jax.experimental.pallas import tpu_sc as plsc

NC = 2
NS = 16
L = 16
NW = NC * NS
C = 128


def _sqrt16(x):
    i = lax.bitcast_convert_type(x, jnp.int32)
    i = jnp.int32(0x5F3759DF) - lax.shift_right_logical(i, 1)
    y = lax.bitcast_convert_type(i, jnp.float32)
    half = x * 0.5
    for _ in range(3):
        y = y * (1.5 - half * y * y)
    return x * y


def _lo(word):
    return lax.bitcast_convert_type(lax.shift_left(word, 16), jnp.float32)


def _hi(word):
    return lax.bitcast_convert_type(
        lax.bitwise_and(word, jnp.int32(-65536)), jnp.float32)


def _pack(x):
    u = lax.bitcast_convert_type(x.astype(jnp.bfloat16), jnp.uint16)
    u = u.astype(jnp.uint32)
    word = u[:, ::2] | (u[:, 1::2] << 16)
    return lax.bitcast_convert_type(word, jnp.int32)


def kernel(h_batch, t_batch, l_batch, h_apos_batch, t_apos_batch,
           l_apos_batch, E, R, W):
    B = h_batch.shape[0]
    D = E.shape[1]
    NR = R.shape[0]
    DP = D // 2
    f32 = jnp.float32
    b_per_w = B // NW
    nchunk = b_per_w // C
    assert b_per_w * NW == B and nchunk * C == b_per_w and D % (2 * L) == 0

    mesh = plsc.VectorSubcoreMesh(core_axis_name="c", subcore_axis_name="s")
    vec = jax.ShapeDtypeStruct((B,), f32)
    matT = jax.ShapeDtypeStruct((D, B), f32)

    idx_t = pltpu.VMEM((C,), jnp.int32)
    row_t = pltpu.VMEM((C, D), f32)
    out_t = pltpu.VMEM((D, C), f32)

    @functools.partial(
        pl.kernel,
        out_type=(vec, vec, matT, matT, matT, matT),
        mesh=mesh,
        compiler_params=pltpu.CompilerParams(
            needs_layout_passes=False, use_tc_tiling_on_sc=False),
        scratch_types=[
            [idx_t] * 2, [idx_t] * 2, [idx_t] * 2,
            [row_t] * 2, [row_t] * 2,
            pltpu.VMEM((NR, DP), jnp.int32),
            pltpu.VMEM((NR, DP), jnp.int32),
            [out_t], [out_t],
            [pltpu.VMEM((C,), f32)],
            pltpu.SemaphoreType.DMA,
            pltpu.SemaphoreType.DMA,
        ],
    )
    def run(h_i, t_i, l_i, ha_i, ta_i, la_i, E_h, R_h, W_h,
            dist_o, dista_o, hp_o, tp_o, hpa_o, tpa_o,
            hi_v, ti_v, li_v, hr, tr, R_t, W_t, hp_v, tp_v, dist_v,
            gsem, ssem):
        cid = lax.axis_index("c")
        sid = lax.axis_index("s")
        wid = sid * NC + cid
        zero = jnp.zeros((L,), f32)

        pltpu.sync_copy(R_h, R_t)
        pltpu.sync_copy(W_h, W_t)

        sides = (
            (h_i, t_i, l_i, dist_o, hp_o, tp_o),
            (ha_i, ta_i, la_i, dista_o, hpa_o, tpa_o),
        )
        tasks = [(s, c) for s in range(2) for c in range(nchunk)]

        def start_gathers(task, slot):
            s, c = task
            hb, tb, lb, _, _, _ = sides[s]
            base = wid * b_per_w + c * C
            pltpu.sync_copy(hb.at[pl.ds(base, C)], hi_v[slot])
            pltpu.sync_copy(tb.at[pl.ds(base, C)], ti_v[slot])
            pltpu.sync_copy(lb.at[pl.ds(base, C)], li_v[slot])
            return [
                pltpu.async_copy(E_h.at[hi_v[slot]], hr[slot], gsem),
                pltpu.async_copy(E_h.at[ti_v[slot]], tr[slot], gsem),
            ]

        def compute(slot):
            hrs, trs, R32, W32 = hr[slot], tr[slot], R_t, W_t
            lis = li_v[slot]
            hps, tps, dv = hp_v[0], tp_v[0], dist_v[0]

            @plsc.parallel_loop(0, C // L)
            def _group(g):
                rows = g * L + lax.iota(jnp.int32, L)
                lv = lis[pl.ds(g * L, L)]

                @plsc.parallel_loop(0, DP, unroll=16, carry=(zero, zero, zero))
                def dots(d, dcarry):
                    n2, sh, st = dcarry
                    col = jnp.full((L,), d, jnp.int32)
                    d2 = 2 * d
                    c0 = jnp.full((L,), 0, jnp.int32) + d2
                    c1 = c0 + 1
                    ww = plsc.load_gather(W32, [lv, col])
                    h0 = plsc.load_gather(hrs, [rows, c0])
                    h1 = plsc.load_gather(hrs, [rows, c1])
                    t0 = plsc.load_gather(trs, [rows, c0])
                    t1 = plsc.load_gather(trs, [rows, c1])
                    w0, w1 = _lo(ww), _hi(ww)
                    return (n2 + w0 * w0 + w1 * w1,
                            sh + h0 * w0 + h1 * w1,
                            st + t0 * w0 + t1 * w1)

                n2, sh, st = dots
                ah = sh / n2
                atc = st / n2

                @plsc.parallel_loop(0, DP, unroll=16, carry=zero)
                def accd(d, acc):
                    col = jnp.full((L,), d, jnp.int32)
                    d2 = 2 * d
                    c0 = jnp.full((L,), 0, jnp.int32) + d2
                    c1 = c0 + 1
                    ww = plsc.load_gather(W32, [lv, col])
                    lw = plsc.load_gather(R32, [lv, col])
                    h0 = plsc.load_gather(hrs, [rows, c0])
                    h1 = plsc.load_gather(hrs, [rows, c1])
                    t0 = plsc.load_gather(trs, [rows, c0])
                    t1 = plsc.load_gather(trs, [rows, c1])
                    w0, w1 = _lo(ww), _hi(ww)
                    l0, l1 = _lo(lw), _hi(lw)
                    hp0 = h0 - ah * w0
                    hp1 = h1 - ah * w1
                    tp0 = t0 - atc * w0
                    tp1 = t1 - atc * w1
                    hps[d2, pl.ds(g * L, L)] = hp0
                    hps[d2 + 1, pl.ds(g * L, L)] = hp1
                    tps[d2, pl.ds(g * L, L)] = tp0
                    tps[d2 + 1, pl.ds(g * L, L)] = tp1
                    q0 = hp0 + l0 - tp0
                    q1 = hp1 + l1 - tp1
                    return acc + q0 * q0 + q1 * q1

                dv[pl.ds(g * L, L)] = _sqrt16(accd)

        def start_stores(task, slot):
            s, c = task
            _, _, _, d_o, hpo, tpo = sides[s]
            base = wid * b_per_w + c * C
            return [
                pltpu.async_copy(hp_v[0], hpo.at[:, pl.ds(base, C)], ssem),
                pltpu.async_copy(tp_v[0], tpo.at[:, pl.ds(base, C)], ssem),
                pltpu.async_copy(dist_v[0], d_o.at[pl.ds(base, C)], ssem),
            ]

        pending_g = start_gathers(tasks[0], 0)
        pending_s = []
        for i, task in enumerate(tasks):
            slot = i % 2
            for cp in pending_g:
                cp.wait()
            if i + 1 < len(tasks):
                pending_g = start_gathers(tasks[i + 1], (i + 1) % 2)
            for cp in pending_s:
                cp.wait()
            compute(slot)
            pending_s = start_stores(task, slot)
        for cp in pending_s:
            cp.wait()

    dist, dist_a, hpT, tpT, hpaT, tpaT = run(
        h_batch.astype(jnp.int32), t_batch.astype(jnp.int32),
        l_batch.astype(jnp.int32), h_apos_batch.astype(jnp.int32),
        t_apos_batch.astype(jnp.int32), l_apos_batch.astype(jnp.int32),
        E, _pack(R), _pack(W))
    return (dist, dist_a, jnp.transpose(hpT), jnp.transpose(tpT),
            jnp.transpose(hpaT), jnp.transpose(tpaT))


# speedup vs baseline: 15.3278x; 1.2067x over previous
import functools

import jax
import jax.numpy as jnp
from jax import lax
from jax.experimental import pallas as pl
from jax.experimental.pallas import tpu as pltpu
from jax.experimental.pallas import tpu_sc as plsc

NC = 2
NS = 16
L = 16
NW = NC * NS
C = 128


def _sqrt16(x):
    i = lax.bitcast_convert_type(x, jnp.int32)
    i = jnp.int32(0x5F3759DF) - lax.shift_right_logical(i, 1)
    y = lax.bitcast_convert_type(i, jnp.float32)
    half = x * 0.5
    for _ in range(3):
        y = y * (1.5 - half * y * y)
    return x * y


def _lo(word):
    return lax.bitcast_convert_type(lax.shift_left(word, 16), jnp.float32)


def _hi(word):
    return lax.bitcast_convert_type(
        lax.bitwise_and(word, jnp.int32(-65536)), jnp.float32)


def _pack(x):
    u = lax.bitcast_convert_type(x.astype(jnp.bfloat16), jnp.uint16)
    u = u.astype(jnp.uint32)
    word = u[:, ::2] | (u[:, 1::2] << 16)
    return lax.bitcast_convert_type(word, jnp.int32)


def kernel(h_batch, t_batch, l_batch, h_apos_batch, t_apos_batch,
           l_apos_batch, E, R, W):
    B = h_batch.shape[0]
    D = E.shape[1]
    NR = R.shape[0]
    DP = D // 2
    f32 = jnp.float32
    b_per_w = B // NW
    nchunk = b_per_w // C
    assert b_per_w * NW == B and nchunk * C == b_per_w and D % (2 * L) == 0

    mesh = plsc.VectorSubcoreMesh(core_axis_name="c", subcore_axis_name="s")
    vec = jax.ShapeDtypeStruct((B,), f32)
    matT = jax.ShapeDtypeStruct((D, B), f32)

    idx_t = pltpu.VMEM((C,), jnp.int32)
    row_t = pltpu.VMEM((C, D), f32)
    out_t = pltpu.VMEM((D, C), f32)

    @functools.partial(
        pl.kernel,
        out_type=(vec, vec, matT, matT, matT, matT),
        mesh=mesh,
        compiler_params=pltpu.CompilerParams(
            needs_layout_passes=False, use_tc_tiling_on_sc=False),
        scratch_types=[
            [idx_t] * 2, [idx_t] * 2, [idx_t] * 2,
            [row_t] * 2, [row_t] * 2,
            pltpu.VMEM((NR, DP), jnp.int32),
            pltpu.VMEM((NR, DP), jnp.int32),
            [out_t], [out_t],
            [pltpu.VMEM((C,), f32)],
            pltpu.SemaphoreType.DMA,
            pltpu.SemaphoreType.DMA,
        ],
    )
    def run(h_i, t_i, l_i, ha_i, ta_i, la_i, E_h, R_h, W_h,
            dist_o, dista_o, hp_o, tp_o, hpa_o, tpa_o,
            hi_v, ti_v, li_v, hr, tr, R_t, W_t, hp_v, tp_v, dist_v,
            gsem, ssem):
        cid = lax.axis_index("c")
        sid = lax.axis_index("s")
        wid = sid * NC + cid
        zero = jnp.zeros((L,), f32)

        pltpu.sync_copy(R_h, R_t)
        pltpu.sync_copy(W_h, W_t)

        sides = (
            (h_i, t_i, l_i, dist_o, hp_o, tp_o),
            (ha_i, ta_i, la_i, dista_o, hpa_o, tpa_o),
        )
        tasks = [(s, c) for s in range(2) for c in range(nchunk)]

        def start_gathers(task, slot):
            s, c = task
            hb, tb, lb, _, _, _ = sides[s]
            base = wid * b_per_w + c * C
            pltpu.sync_copy(hb.at[pl.ds(base, C)], hi_v[slot])
            pltpu.sync_copy(tb.at[pl.ds(base, C)], ti_v[slot])
            pltpu.sync_copy(lb.at[pl.ds(base, C)], li_v[slot])
            return [
                pltpu.async_copy(E_h.at[hi_v[slot]], hr[slot], gsem),
                pltpu.async_copy(E_h.at[ti_v[slot]], tr[slot], gsem),
            ]

        def compute(slot):
            hrs, trs, R32, W32 = hr[slot], tr[slot], R_t, W_t
            lis = li_v[slot]
            hps, tps, dv = hp_v[0], tp_v[0], dist_v[0]

            @plsc.parallel_loop(0, C // L)
            def _group(g):
                iota = lax.iota(jnp.int32, L)
                rows = g * L + iota
                lv = lis[pl.ds(g * L, L)]

                @plsc.parallel_loop(0, DP, unroll=8, carry=(zero, zero, zero))
                def dots(d, dcarry):
                    n2, sh, st = dcarry
                    pv = (jnp.full((L,), 0, jnp.int32) + d + iota) & (DP - 1)
                    c0 = 2 * pv
                    c1 = c0 + 1
                    ww = plsc.load_gather(W32, [lv, pv])
                    h0 = plsc.load_gather(hrs, [rows, c0])
                    h1 = plsc.load_gather(hrs, [rows, c1])
                    t0 = plsc.load_gather(trs, [rows, c0])
                    t1 = plsc.load_gather(trs, [rows, c1])
                    w0, w1 = _lo(ww), _hi(ww)
                    return (n2 + w0 * w0 + w1 * w1,
                            sh + h0 * w0 + h1 * w1,
                            st + t0 * w0 + t1 * w1)

                n2, sh, st = dots
                ah = sh / n2
                atc = st / n2

                @plsc.parallel_loop(0, DP, unroll=8, carry=zero)
                def accd(d, acc):
                    pv = (jnp.full((L,), 0, jnp.int32) + d + iota) & (DP - 1)
                    c0 = 2 * pv
                    c1 = c0 + 1
                    ww = plsc.load_gather(W32, [lv, pv])
                    lw = plsc.load_gather(R32, [lv, pv])
                    h0 = plsc.load_gather(hrs, [rows, c0])
                    h1 = plsc.load_gather(hrs, [rows, c1])
                    t0 = plsc.load_gather(trs, [rows, c0])
                    t1 = plsc.load_gather(trs, [rows, c1])
                    w0, w1 = _lo(ww), _hi(ww)
                    l0, l1 = _lo(lw), _hi(lw)
                    hp0 = h0 - ah * w0
                    hp1 = h1 - ah * w1
                    tp0 = t0 - atc * w0
                    tp1 = t1 - atc * w1
                    plsc.store_scatter(hps, [c0, rows], hp0)
                    plsc.store_scatter(hps, [c1, rows], hp1)
                    plsc.store_scatter(tps, [c0, rows], tp0)
                    plsc.store_scatter(tps, [c1, rows], tp1)
                    q0 = hp0 + l0 - tp0
                    q1 = hp1 + l1 - tp1
                    return acc + q0 * q0 + q1 * q1

                dv[pl.ds(g * L, L)] = _sqrt16(accd)

        def start_stores(task, slot):
            s, c = task
            _, _, _, d_o, hpo, tpo = sides[s]
            base = wid * b_per_w + c * C
            return [
                pltpu.async_copy(hp_v[0], hpo.at[:, pl.ds(base, C)], ssem),
                pltpu.async_copy(tp_v[0], tpo.at[:, pl.ds(base, C)], ssem),
                pltpu.async_copy(dist_v[0], d_o.at[pl.ds(base, C)], ssem),
            ]

        pending_g = start_gathers(tasks[0], 0)
        pending_s = []
        for i, task in enumerate(tasks):
            slot = i % 2
            for cp in pending_g:
                cp.wait()
            if i + 1 < len(tasks):
                pending_g = start_gathers(tasks[i + 1], (i + 1) % 2)
            for cp in pending_s:
                cp.wait()
            compute(slot)
            pending_s = start_stores(task, slot)
        for cp in pending_s:
            cp.wait()

    dist, dist_a, hpT, tpT, hpaT, tpaT = run(
        h_batch.astype(jnp.int32), t_batch.astype(jnp.int32),
        l_batch.astype(jnp.int32), h_apos_batch.astype(jnp.int32),
        t_apos_batch.astype(jnp.int32), l_apos_batch.astype(jnp.int32),
        E, _pack(R), _pack(W))
    return (dist, dist_a, jnp.transpose(hpT), jnp.transpose(tpT),
            jnp.transpose(hpaT), jnp.transpose(tpaT))
